# Initial kernel scaffold; baseline (speedup 1.0000x reference)
import jax
import jax.numpy as jnp
from jax.experimental import pallas as pl


def kernel(feature_nodes, feature_edges, edge_index, W_self1, W_neigh1, b1, W_self2, W_neigh2, b2):
    raise NotImplementedError("write your pallas kernel here")


# speedup vs baseline: 4.5659x; 4.5659x over previous
import functools

import jax
import jax.numpy as jnp
from jax import lax
from jax.experimental import pallas as pl
from jax.experimental.pallas import tpu as pltpu
from jax.experimental.pallas import tpu_sc as plsc

N = 10000
E = 320000
D_FEAT = 128
D_EDGE = 16
IN_FEATS = D_FEAT + D_EDGE
H_FEATS = 128

NC = 2
NS = 16
NW = NC * NS
EB = 80
E_PER_W = E // NW
NBLK = E_PER_W // EB
N_CHUNK = 624
N_TAIL = N - NS * N_CHUNK

_SC_MESH = plsc.VectorSubcoreMesh(core_axis_name="c", subcore_axis_name="s")


def _zero_fill(zb, nrows, ncols):
    zrow = jnp.zeros((16,), jnp.float32)

    def body(i, _):
        for c in range(ncols // 16):
            zb[i, pl.ds(c * 16, 16)] = zrow
        return 0

    lax.fori_loop(0, nrows, body, 0)


ZB_ROWS = 16


def _zero_acc(zb, acc, sid):
    base = sid * N_CHUNK

    def body(k, _):
        pltpu.sync_copy(zb, acc.at[pl.ds(base + k * ZB_ROWS, ZB_ROWS)])
        return 0

    lax.fori_loop(0, N_CHUNK // ZB_ROWS, body, 0)

    @pl.when(sid == 0)
    def _():
        pltpu.sync_copy(zb, acc.at[pl.ds(NS * N_CHUNK, N_TAIL)])


def _write_acc(acc, out, cid, sid):
    base = sid * N_CHUNK
    pltpu.sync_copy(acc.at[pl.ds(base, N_CHUNK)],
                    out.at[cid, pl.ds(base, N_CHUNK)])

    @pl.when(sid == 0)
    def _():
        pltpu.sync_copy(acc.at[pl.ds(NS * N_CHUNK, N_TAIL)],
                        out.at[cid, pl.ds(NS * N_CHUNK, N_TAIL)])


@functools.partial(
    pl.kernel,
    mesh=_SC_MESH,
    compiler_params=pltpu.CompilerParams(use_tc_tiling_on_sc=False),
    out_type=(
        jax.ShapeDtypeStruct((NC, N, D_EDGE), jnp.float32),
        jax.ShapeDtypeStruct((NC, N, D_EDGE), jnp.float32),
    ),
    scratch_types=[
        pltpu.VMEM((EB,), jnp.int32),
        pltpu.VMEM((EB, D_EDGE), jnp.float32),
        pltpu.VMEM((EB, D_EDGE), jnp.float32),
        pltpu.VMEM((ZB_ROWS, D_EDGE), jnp.float32),
        pltpu.VMEM_SHARED((N, D_EDGE), jnp.float32),
        pltpu.VMEM_SHARED((N, D_EDGE), jnp.float32),
    ],
)
def _sc_edge_mean(fe_hbm, dst_hbm, esum_out, deg_out,
                  dst_v, val_v, ones_v, zb, acc_e, acc_d):
    cid = lax.axis_index("c")
    sid = lax.axis_index("s")
    wid = sid * NC + cid
    ebase = wid * E_PER_W

    one = jnp.ones((16,), jnp.float32)

    def fill_ones(i, _):
        ones_v[i, :] = one
        return 0

    lax.fori_loop(0, EB, fill_ones, 0)
    _zero_fill(zb, ZB_ROWS, D_EDGE)

    _zero_acc(zb, acc_e, sid)
    _zero_acc(zb, acc_d, sid)
    plsc.subcore_barrier()

    def step(i, _):
        off = ebase + i * EB
        pltpu.sync_copy(dst_hbm.at[pl.ds(off, EB)], dst_v)
        pltpu.sync_copy(fe_hbm.at[pl.ds(off, EB)], val_v)
        pltpu.sync_copy(val_v, acc_e.at[dst_v], add=True)
        pltpu.sync_copy(ones_v, acc_d.at[dst_v], add=True)
        return 0

    lax.fori_loop(0, NBLK, step, 0)
    plsc.subcore_barrier()

    _write_acc(acc_e, esum_out, cid, sid)
    _write_acc(acc_d, deg_out, cid, sid)


@functools.partial(
    pl.kernel,
    mesh=_SC_MESH,
    out_type=jax.ShapeDtypeStruct((NC, N, H_FEATS), jnp.float32),
    scratch_types=[
        pltpu.VMEM((EB,), jnp.int32),
        pltpu.VMEM((EB,), jnp.int32),
        pltpu.VMEM((EB, H_FEATS), jnp.float32),
        pltpu.VMEM((ZB_ROWS, H_FEATS), jnp.float32),
        pltpu.VMEM_SHARED((N, H_FEATS), jnp.float32),
        pltpu.SemaphoreType.DMA,
    ],
)
def _sc_agg(z_hbm, src_hbm, dst_hbm, s_out,
            src_v, dst_v, rows_v, zb, acc, sem):
    cid = lax.axis_index("c")
    sid = lax.axis_index("s")
    wid = sid * NC + cid
    ebase = wid * E_PER_W

    _zero_fill(zb, ZB_ROWS, H_FEATS)
    _zero_acc(zb, acc, sid)
    plsc.subcore_barrier()

    def step(i, _):
        off = ebase + i * EB
        pltpu.sync_copy(src_hbm.at[pl.ds(off, EB)], src_v)
        pltpu.sync_copy(dst_hbm.at[pl.ds(off, EB)], dst_v)
        pltpu.async_copy(z_hbm.at[src_v], rows_v, sem).wait()
        pltpu.sync_copy(rows_v, acc.at[dst_v], add=True)
        return 0

    lax.fori_loop(0, NBLK, step, 0)
    plsc.subcore_barrier()

    _write_acc(acc, s_out, cid, sid)


_RB = 2000
_GRID = N // _RB


def _stage_a_body(fn_ref, e_ref, d_ref, w_ref, h_ref, z_ref, deg_ref):
    d = d_ref[0] + d_ref[1]
    deg = jnp.maximum(d[:, 0:1], 1.0)
    he = (e_ref[0] + e_ref[1]) / deg
    h = jnp.concatenate([fn_ref[...], he], axis=1)
    h_ref[...] = h
    z_ref[...] = jnp.dot(h, w_ref[...], preferred_element_type=jnp.float32)
    deg_ref[...] = deg


def _stage_a(fn, esum, dsum, w_neigh1):
    return pl.pallas_call(
        _stage_a_body,
        grid=(_GRID,),
        in_specs=[
            pl.BlockSpec((_RB, D_FEAT), lambda i: (i, 0)),
            pl.BlockSpec((NC, _RB, D_EDGE), lambda i: (0, i, 0)),
            pl.BlockSpec((NC, _RB, D_EDGE), lambda i: (0, i, 0)),
            pl.BlockSpec((IN_FEATS, H_FEATS), lambda i: (0, 0)),
        ],
        out_specs=[
            pl.BlockSpec((_RB, IN_FEATS), lambda i: (i, 0)),
            pl.BlockSpec((_RB, H_FEATS), lambda i: (i, 0)),
            pl.BlockSpec((_RB, 1), lambda i: (i, 0)),
        ],
        out_shape=[
            jax.ShapeDtypeStruct((N, IN_FEATS), jnp.float32),
            jax.ShapeDtypeStruct((N, H_FEATS), jnp.float32),
            jax.ShapeDtypeStruct((N, 1), jnp.float32),
        ],
    )(fn, esum, dsum, w_neigh1)


def _stage_b_body(h_ref, s_ref, deg_ref, ws_ref, b_ref, wn_ref,
                  h1_ref, z2_ref):
    agg = (s_ref[0] + s_ref[1]) / deg_ref[...]
    h1 = jnp.dot(h_ref[...], ws_ref[...],
                 preferred_element_type=jnp.float32) + agg + b_ref[...]
    h1 = jnp.maximum(h1, 0.0)
    h1_ref[...] = h1
    z2_ref[...] = jnp.dot(h1, wn_ref[...], preferred_element_type=jnp.float32)


def _stage_b(h, s1, deg, w_self1, b1, w_neigh2):
    return pl.pallas_call(
        _stage_b_body,
        grid=(_GRID,),
        in_specs=[
            pl.BlockSpec((_RB, IN_FEATS), lambda i: (i, 0)),
            pl.BlockSpec((NC, _RB, H_FEATS), lambda i: (0, i, 0)),
            pl.BlockSpec((_RB, 1), lambda i: (i, 0)),
            pl.BlockSpec((IN_FEATS, H_FEATS), lambda i: (0, 0)),
            pl.BlockSpec((1, H_FEATS), lambda i: (0, 0)),
            pl.BlockSpec((H_FEATS, H_FEATS), lambda i: (0, 0)),
        ],
        out_specs=[
            pl.BlockSpec((_RB, H_FEATS), lambda i: (i, 0)),
            pl.BlockSpec((_RB, H_FEATS), lambda i: (i, 0)),
        ],
        out_shape=[
            jax.ShapeDtypeStruct((N, H_FEATS), jnp.float32),
            jax.ShapeDtypeStruct((N, H_FEATS), jnp.float32),
        ],
    )(h, s1, deg, w_self1, b1, w_neigh2)


def _stage_c_body(h1_ref, s_ref, deg_ref, ws_ref, b_ref, out_ref):
    agg = (s_ref[0] + s_ref[1]) / deg_ref[...]
    out_ref[...] = (jnp.dot(h1_ref[...], ws_ref[...],
                            preferred_element_type=jnp.float32)
                    + agg + b_ref[...])


def _stage_c(h1, s2, deg, w_self2, b2):
    return pl.pallas_call(
        _stage_c_body,
        grid=(_GRID,),
        in_specs=[
            pl.BlockSpec((_RB, H_FEATS), lambda i: (i, 0)),
            pl.BlockSpec((NC, _RB, H_FEATS), lambda i: (0, i, 0)),
            pl.BlockSpec((_RB, 1), lambda i: (i, 0)),
            pl.BlockSpec((H_FEATS, H_FEATS), lambda i: (0, 0)),
            pl.BlockSpec((1, H_FEATS), lambda i: (0, 0)),
        ],
        out_specs=pl.BlockSpec((_RB, H_FEATS), lambda i: (i, 0)),
        out_shape=jax.ShapeDtypeStruct((N, H_FEATS), jnp.float32),
    )(h1, s2, deg, w_self2, b2)


def kernel(feature_nodes, feature_edges, edge_index, W_self1, W_neigh1, b1,
           W_self2, W_neigh2, b2):
    src = edge_index[0]
    dst = edge_index[1]
    b1r = b1.reshape(1, H_FEATS)
    b2r = b2.reshape(1, H_FEATS)

    esum, dsum = _sc_edge_mean(feature_edges, dst)
    h, z1, deg = _stage_a(feature_nodes, esum, dsum, W_neigh1)
    s1 = _sc_agg(z1, src, dst)
    h1, z2 = _stage_b(h, s1, deg, W_self1, b1r, W_neigh2)
    s2 = _sc_agg(z2, src, dst)
    return _stage_c(h1, s2, deg, W_self2, b2r)
